# Initial kernel scaffold; baseline (speedup 1.0000x reference)
import jax
import jax.numpy as jnp
from jax.experimental import pallas as pl


def kernel(xyz, points, fps_idx, W0, g0, b0, W1, g1, b1, W2, g2, b2):
    raise NotImplementedError("write your pallas kernel here")


# speedup vs baseline: 9.6546x; 9.6546x over previous
import functools

import jax
import jax.numpy as jnp
from jax import lax
from jax.experimental import pallas as pl
from jax.experimental.pallas import tpu as pltpu
from jax.experimental.pallas import tpu_sc as plsc

B, N, S, NS, D = 8, 8192, 1024, 32, 64
C_IN = 3 + D
CP = 128
MLP_CH = (64, 64, 128)
EPS = 1e-5
M_SAMPLES = float(B * S * NS)

S_BLK = 128
G_SBLK = 128
MLP_TILES = (B * S) // G_SBLK

_NC, _NSUB = 2, 16
_NW = _NC * _NSUB


def _sc_gather_body(n_rows, window, table_hbm, idx_hbm, out_hbm, idx_v, rows_v, sem):
    per_w = n_rows // _NW
    wid = lax.axis_index("s") * _NC + lax.axis_index("c")
    base = wid * per_w
    for w in range(per_w // window):
        off = base + w * window
        pltpu.sync_copy(idx_hbm.at[pl.ds(off, window)], idx_v)
        pltpu.async_copy(table_hbm.at[idx_v], rows_v, sem).wait()
        pltpu.sync_copy(rows_v, out_hbm.at[pl.ds(off, window)])


def _sc_gather(table, idx, window):
    n_rows = idx.shape[0]
    mesh = plsc.VectorSubcoreMesh(core_axis_name="c", subcore_axis_name="s")
    fn = pl.kernel(
        functools.partial(_sc_gather_body, n_rows, window),
        out_type=jax.ShapeDtypeStruct((n_rows, CP), jnp.float32),
        mesh=mesh,
        scratch_types=[
            pltpu.VMEM((window,), jnp.int32),
            pltpu.VMEM((window, CP), jnp.float32),
            pltpu.SemaphoreType.DMA,
        ],
    )
    return fn(table, idx)


def _d2_topk_kernel(cg_ref, xyz_ref, out_ref):
    b = pl.program_id(0)
    qx = cg_ref[0, :, 0:1]
    qy = cg_ref[0, :, 1:2]
    qz = cg_ref[0, :, 2:3]
    rx = xyz_ref[0, 0:1, :]
    ry = xyz_ref[0, 1:2, :]
    rz = xyz_ref[0, 2:3, :]
    rr = rx * rx + ry * ry + rz * rz
    qq = qx * qx + qy * qy + qz * qz
    q3 = cg_ref[0, :, 0:3].astype(jnp.bfloat16)
    r3 = xyz_ref[0, 0:3, :].astype(jnp.bfloat16)
    e = jnp.dot(q3, r3, preferred_element_type=jnp.float32)
    d2 = (qq - 2.0 * e) + rr
    iota = lax.broadcasted_iota(jnp.int32, (S_BLK, N), 1)
    big = jnp.float32(3.0e38)
    for k in range(NS):
        v = jnp.min(d2, axis=1, keepdims=True)
        m = d2 == v
        i = jnp.min(jnp.where(m, iota, N), axis=1, keepdims=True)
        out_ref[0, :, k : k + 1] = i + b * N
        d2 = jnp.where(iota == i, big, d2)


def _d2_topk(cg, xyz_pad):
    cg3 = cg.reshape(B, S, CP)
    grid = (B, S // S_BLK)
    return pl.pallas_call(
        _d2_topk_kernel,
        grid=grid,
        in_specs=[
            pl.BlockSpec((1, S_BLK, CP), lambda b, t: (b, t, 0)),
            pl.BlockSpec((1, 8, N), lambda b, t: (b, 0, 0)),
        ],
        out_specs=pl.BlockSpec((1, S_BLK, NS), lambda b, t: (b, t, 0)),
        out_shape=jax.ShapeDtypeStruct((B, S, NS), jnp.int32),
    )(cg3, xyz_pad)


def _bn_coeffs(acc_sum, acc_sq, params, l, nch):
    mean = acc_sum[l, 0:nch] * (1.0 / M_SAMPLES)
    var = acc_sq[l, 0:nch] * (1.0 / M_SAMPLES) - mean * mean
    var = jnp.maximum(var, 0.0)
    g = params[2 * l, 0:nch]
    bb = params[2 * l + 1, 0:nch]
    a = g * lax.rsqrt(var + EPS)
    c = bb - mean * a
    return a.reshape(1, nch), c.reshape(1, nch)


def _accum(acc_sum, acc_sq, y, l, nch):
    acc_sum[l, 0:nch] += jnp.sum(y, axis=0)
    acc_sq[l, 0:nch] += jnp.sum(y * y, axis=0)


def _mlp_kernel(g3_ref, cg_ref, w0_ref, w1_ref, w2_ref, wx_ref, par_ref,
                out_ref, acc_sum, acc_sq):
    phase = pl.program_id(0)
    t = pl.program_id(1)

    @pl.when(jnp.logical_and(phase == 0, t == 0))
    def _init():
        acc_sum[...] = jnp.zeros_like(acc_sum)
        acc_sq[...] = jnp.zeros_like(acc_sq)

    x = g3_ref[...].reshape(G_SBLK * NS, CP)
    qx = cg_ref[:, 0:1]
    qy = cg_ref[:, 1:2]
    qz = cg_ref[:, 2:3]
    offs = (qx * wx_ref[0:1, :] + qy * wx_ref[1:2, :] + qz * wx_ref[2:3, :])
    offs = jnp.broadcast_to(offs[:, None, :], (G_SBLK, NS, MLP_CH[0]))
    offs = offs.reshape(G_SBLK * NS, MLP_CH[0])

    def y1_of():
        y = jnp.dot(x, w0_ref[...], preferred_element_type=jnp.float32)
        return y - offs

    def z_of(y, l, nch):
        a, c = _bn_coeffs(acc_sum, acc_sq, par_ref, l, nch)
        return jnp.maximum(y * a + c, 0.0)

    @pl.when(phase == 0)
    def _p0():
        _accum(acc_sum, acc_sq, y1_of(), 0, MLP_CH[0])

    @pl.when(phase == 1)
    def _p1():
        z1 = z_of(y1_of(), 0, MLP_CH[0])
        y2 = jnp.dot(z1, w1_ref[...], preferred_element_type=jnp.float32)
        _accum(acc_sum, acc_sq, y2, 1, MLP_CH[1])

    @pl.when(phase == 2)
    def _p2():
        z1 = z_of(y1_of(), 0, MLP_CH[0])
        y2 = jnp.dot(z1, w1_ref[...], preferred_element_type=jnp.float32)
        z2 = z_of(y2, 1, MLP_CH[1])
        y3 = jnp.dot(z2, w2_ref[...], preferred_element_type=jnp.float32)
        _accum(acc_sum, acc_sq, y3, 2, MLP_CH[2])

    @pl.when(phase == 3)
    def _p3():
        z1 = z_of(y1_of(), 0, MLP_CH[0])
        y2 = jnp.dot(z1, w1_ref[...], preferred_element_type=jnp.float32)
        z2 = z_of(y2, 1, MLP_CH[1])
        y3 = jnp.dot(z2, w2_ref[...], preferred_element_type=jnp.float32)
        z3 = z_of(y3, 2, MLP_CH[2])
        pooled = jnp.max(z3.reshape(G_SBLK, NS, MLP_CH[2]), axis=1)
        out_ref[...] = pooled


def _mlp(g3, cg, w0p, w1p, w2p, wxyz, params):
    grid = (4, MLP_TILES)
    full = lambda *_: tuple(0 for _ in range(2))
    return pl.pallas_call(
        _mlp_kernel,
        grid=grid,
        in_specs=[
            pl.BlockSpec((G_SBLK, NS, CP), lambda p, t: (t, 0, 0)),
            pl.BlockSpec((G_SBLK, CP), lambda p, t: (t, 0)),
            pl.BlockSpec((CP, MLP_CH[0]), full),
            pl.BlockSpec((MLP_CH[0], MLP_CH[1]), full),
            pl.BlockSpec((MLP_CH[1], MLP_CH[2]), full),
            pl.BlockSpec((8, MLP_CH[0]), full),
            pl.BlockSpec((8, 128), full),
        ],
        out_specs=pl.BlockSpec((G_SBLK, MLP_CH[2]), lambda p, t: (t, 0)),
        out_shape=jax.ShapeDtypeStruct((B * S, MLP_CH[2]), jnp.float32),
        scratch_shapes=[
            pltpu.VMEM((8, 128), jnp.float32),
            pltpu.VMEM((8, 128), jnp.float32),
        ],
    )(g3, cg, w0p, w1p, w2p, wxyz, params)


def kernel(xyz, points, fps_idx, W0, g0, b0, W1, g1, b1, W2, g2, b2):
    f32 = jnp.float32
    table = jnp.concatenate([xyz, points], axis=1)
    table = jnp.transpose(table, (0, 2, 1)).reshape(B * N, C_IN)
    table = jnp.pad(table, ((0, 0), (0, CP - C_IN)))

    fps_glob = (fps_idx + (jnp.arange(B, dtype=jnp.int32) * N)[:, None]).reshape(-1)

    cg = _sc_gather(table, fps_glob, 256)

    xyz_pad = jnp.pad(xyz, ((0, 0), (0, 5), (0, 0)))
    knn_glob = _d2_topk(cg, xyz_pad)

    g_rows = _sc_gather(table, knn_glob.reshape(-1), 512)
    g3 = g_rows.reshape(B * S, NS, CP)

    w0p = jnp.pad(W0, ((0, 0), (0, CP - C_IN))).T.astype(f32)
    w1p = W1.T.astype(f32)
    w2p = W2.T.astype(f32)
    wxyz = jnp.pad(W0[:, 0:3].T, ((0, 5), (0, 0))).astype(f32)
    params = jnp.zeros((8, 128), f32)
    params = params.at[0, 0:64].set(g0).at[1, 0:64].set(b0)
    params = params.at[2, 0:64].set(g1).at[3, 0:64].set(b1)
    params = params.at[4, 0:128].set(g2).at[5, 0:128].set(b2)

    pooled = _mlp(g3, cg, w0p, w1p, w2p, wxyz, params)

    new_xyz = jnp.transpose(cg[:, 0:3].reshape(B, S, 3), (0, 2, 1))
    new_points = jnp.transpose(pooled.reshape(B, S, MLP_CH[2]), (0, 2, 1))
    return (new_xyz, new_points, fps_idx)


# speedup vs baseline: 16.1094x; 1.6686x over previous
import functools

import jax
import jax.numpy as jnp
from jax import lax
from jax.experimental import pallas as pl
from jax.experimental.pallas import tpu as pltpu
from jax.experimental.pallas import tpu_sc as plsc

B, N, S, NS, D = 8, 8192, 1024, 32, 64
C_IN = 3 + D
CP = 128
MLP_CH = (64, 64, 128)
EPS = 1e-5
M_SAMPLES = float(B * S * NS)

S_BLK = 128
G_SBLK = 128
MLP_TILES = (B * S) // G_SBLK

_NC, _NSUB = 2, 16
_NW = _NC * _NSUB


def _sc_gather_body(n_rows, window, table_hbm, idx_hbm, out_hbm, idx_v, rows_v, sem):
    per_w = n_rows // _NW
    wid = lax.axis_index("s") * _NC + lax.axis_index("c")
    base = wid * per_w
    for w in range(per_w // window):
        off = base + w * window
        pltpu.sync_copy(idx_hbm.at[pl.ds(off, window)], idx_v)
        pltpu.async_copy(table_hbm.at[idx_v], rows_v, sem).wait()
        pltpu.sync_copy(rows_v, out_hbm.at[pl.ds(off, window)])


def _sc_gather(table, idx, window):
    n_rows = idx.shape[0]
    mesh = plsc.VectorSubcoreMesh(core_axis_name="c", subcore_axis_name="s")
    fn = pl.kernel(
        functools.partial(_sc_gather_body, n_rows, window),
        out_type=jax.ShapeDtypeStruct((n_rows, CP), jnp.float32),
        mesh=mesh,
        scratch_types=[
            pltpu.VMEM((window,), jnp.int32),
            pltpu.VMEM((window, CP), jnp.float32),
            pltpu.SemaphoreType.DMA,
        ],
    )
    return fn(table, idx)


_CH = N // 128
_TOPQ = 6


def _compute_d2(cg_ref, xyz_ref):
    qx = cg_ref[0, :, 0:1]
    qy = cg_ref[0, :, 1:2]
    qz = cg_ref[0, :, 2:3]
    rx = xyz_ref[0, 0:1, :]
    ry = xyz_ref[0, 1:2, :]
    rz = xyz_ref[0, 2:3, :]
    rr = rx * rx + ry * ry + rz * rz
    qq = qx * qx + qy * qy + qz * qz
    q3 = cg_ref[0, :, 0:3].astype(jnp.bfloat16)
    r3 = xyz_ref[0, 0:3, :].astype(jnp.bfloat16)
    e = jnp.dot(q3, r3, preferred_element_type=jnp.float32)
    return (qq - 2.0 * e) + rr


def _d2_topk_kernel(cg_ref, xyz_ref, out_ref):
    b = pl.program_id(0)
    big = jnp.float32(3.0e38)
    d2 = _compute_d2(cg_ref, xyz_ref)

    d2v = d2.reshape(S_BLK, _CH, 128)
    jio = lax.broadcasted_iota(jnp.int32, (S_BLK, _CH, 128), 1)
    ms, js = [], []
    for t in range(_TOPQ):
        m = jnp.min(d2v, axis=1)
        jt = jnp.min(jnp.where(d2v == m[:, None, :], jio, _CH), axis=1)
        ms.append(m)
        js.append(jt)
        if t < _TOPQ - 1:
            d2v = jnp.where(jio == jt[:, None, :], big, d2v)

    lane = lax.broadcasted_iota(jnp.int32, (S_BLK, 128), 1)
    invacc = jnp.zeros((S_BLK, 1), jnp.float32)
    for k in range(NS):
        v = jnp.min(ms[0], axis=1, keepdims=True)
        nh = js[0] * 128 + lane
        n = jnp.min(jnp.where(ms[0] == v, nh, N), axis=1, keepdims=True)
        out_ref[0, :, k : k + 1] = n + b * N
        sel = lane == lax.rem(n, 128)
        for t in range(_TOPQ - 1):
            ms[t] = jnp.where(sel, ms[t + 1], ms[t])
            js[t] = jnp.where(sel, js[t + 1], js[t])
        ms[_TOPQ - 1] = jnp.where(sel, big, ms[_TOPQ - 1])
        if k < NS - 1:
            hit = jnp.max(jnp.where(sel, ms[0], 0.0), axis=1, keepdims=True)
            invacc = jnp.maximum(invacc, hit)

    @pl.when(jnp.max(invacc) >= big)
    def _slow_path():
        d2s = _compute_d2(cg_ref, xyz_ref)
        iota = lax.broadcasted_iota(jnp.int32, (S_BLK, N), 1)
        for k in range(NS):
            v = jnp.min(d2s, axis=1, keepdims=True)
            i = jnp.min(jnp.where(d2s == v, iota, N), axis=1, keepdims=True)
            out_ref[0, :, k : k + 1] = i + b * N
            d2s = jnp.where(iota == i, big, d2s)


def _d2_topk(cg, xyz_pad):
    cg3 = cg.reshape(B, S, CP)
    grid = (B, S // S_BLK)
    return pl.pallas_call(
        _d2_topk_kernel,
        grid=grid,
        in_specs=[
            pl.BlockSpec((1, S_BLK, CP), lambda b, t: (b, t, 0)),
            pl.BlockSpec((1, 8, N), lambda b, t: (b, 0, 0)),
        ],
        out_specs=pl.BlockSpec((1, S_BLK, NS), lambda b, t: (b, t, 0)),
        out_shape=jax.ShapeDtypeStruct((B, S, NS), jnp.int32),
    )(cg3, xyz_pad)


def _bn_coeffs(acc_sum, acc_sq, params, l, nch):
    mean = acc_sum[l, 0:nch] * (1.0 / M_SAMPLES)
    var = acc_sq[l, 0:nch] * (1.0 / M_SAMPLES) - mean * mean
    var = jnp.maximum(var, 0.0)
    g = params[2 * l, 0:nch]
    bb = params[2 * l + 1, 0:nch]
    a = g * lax.rsqrt(var + EPS)
    c = bb - mean * a
    return a.reshape(1, nch), c.reshape(1, nch)


def _accum(acc_sum, acc_sq, y, l, nch):
    acc_sum[l, 0:nch] += jnp.sum(y, axis=0)
    acc_sq[l, 0:nch] += jnp.sum(y * y, axis=0)


def _mlp_kernel(g3_ref, cg_ref, w0_ref, w1_ref, w2_ref, wx_ref, par_ref,
                out_ref, acc_sum, acc_sq):
    phase = pl.program_id(0)
    t = pl.program_id(1)

    @pl.when(jnp.logical_and(phase == 0, t == 0))
    def _init():
        acc_sum[...] = jnp.zeros_like(acc_sum)
        acc_sq[...] = jnp.zeros_like(acc_sq)

    x = g3_ref[...].reshape(G_SBLK * NS, CP)
    qx = cg_ref[:, 0:1]
    qy = cg_ref[:, 1:2]
    qz = cg_ref[:, 2:3]
    offs = (qx * wx_ref[0:1, :] + qy * wx_ref[1:2, :] + qz * wx_ref[2:3, :])
    offs = jnp.broadcast_to(offs[:, None, :], (G_SBLK, NS, MLP_CH[0]))
    offs = offs.reshape(G_SBLK * NS, MLP_CH[0])

    def y1_of():
        y = jnp.dot(x, w0_ref[...], preferred_element_type=jnp.float32)
        return y - offs

    def z_of(y, l, nch):
        a, c = _bn_coeffs(acc_sum, acc_sq, par_ref, l, nch)
        return jnp.maximum(y * a + c, 0.0)

    @pl.when(phase == 0)
    def _p0():
        _accum(acc_sum, acc_sq, y1_of(), 0, MLP_CH[0])

    @pl.when(phase == 1)
    def _p1():
        z1 = z_of(y1_of(), 0, MLP_CH[0])
        y2 = jnp.dot(z1, w1_ref[...], preferred_element_type=jnp.float32)
        _accum(acc_sum, acc_sq, y2, 1, MLP_CH[1])

    @pl.when(phase == 2)
    def _p2():
        z1 = z_of(y1_of(), 0, MLP_CH[0])
        y2 = jnp.dot(z1, w1_ref[...], preferred_element_type=jnp.float32)
        z2 = z_of(y2, 1, MLP_CH[1])
        y3 = jnp.dot(z2, w2_ref[...], preferred_element_type=jnp.float32)
        _accum(acc_sum, acc_sq, y3, 2, MLP_CH[2])

    @pl.when(phase == 3)
    def _p3():
        z1 = z_of(y1_of(), 0, MLP_CH[0])
        y2 = jnp.dot(z1, w1_ref[...], preferred_element_type=jnp.float32)
        z2 = z_of(y2, 1, MLP_CH[1])
        y3 = jnp.dot(z2, w2_ref[...], preferred_element_type=jnp.float32)
        z3 = z_of(y3, 2, MLP_CH[2])
        pooled = jnp.max(z3.reshape(G_SBLK, NS, MLP_CH[2]), axis=1)
        out_ref[...] = pooled


def _mlp(g3, cg, w0p, w1p, w2p, wxyz, params):
    grid = (4, MLP_TILES)
    full = lambda *_: tuple(0 for _ in range(2))
    return pl.pallas_call(
        _mlp_kernel,
        grid=grid,
        in_specs=[
            pl.BlockSpec((G_SBLK, NS, CP), lambda p, t: (t, 0, 0)),
            pl.BlockSpec((G_SBLK, CP), lambda p, t: (t, 0)),
            pl.BlockSpec((CP, MLP_CH[0]), full),
            pl.BlockSpec((MLP_CH[0], MLP_CH[1]), full),
            pl.BlockSpec((MLP_CH[1], MLP_CH[2]), full),
            pl.BlockSpec((8, MLP_CH[0]), full),
            pl.BlockSpec((8, 128), full),
        ],
        out_specs=pl.BlockSpec((G_SBLK, MLP_CH[2]), lambda p, t: (t, 0)),
        out_shape=jax.ShapeDtypeStruct((B * S, MLP_CH[2]), jnp.float32),
        scratch_shapes=[
            pltpu.VMEM((8, 128), jnp.float32),
            pltpu.VMEM((8, 128), jnp.float32),
        ],
    )(g3, cg, w0p, w1p, w2p, wxyz, params)


def kernel(xyz, points, fps_idx, W0, g0, b0, W1, g1, b1, W2, g2, b2):
    f32 = jnp.float32
    table = jnp.concatenate([xyz, points], axis=1)
    table = jnp.transpose(table, (0, 2, 1)).reshape(B * N, C_IN)
    table = jnp.pad(table, ((0, 0), (0, CP - C_IN)))

    fps_glob = (fps_idx + (jnp.arange(B, dtype=jnp.int32) * N)[:, None]).reshape(-1)

    cg = _sc_gather(table, fps_glob, 256)

    xyz_pad = jnp.pad(xyz, ((0, 0), (0, 5), (0, 0)))
    knn_glob = _d2_topk(cg, xyz_pad)

    g_rows = _sc_gather(table, knn_glob.reshape(-1), 512)
    g3 = g_rows.reshape(B * S, NS, CP)

    w0p = jnp.pad(W0, ((0, 0), (0, CP - C_IN))).T.astype(f32)
    w1p = W1.T.astype(f32)
    w2p = W2.T.astype(f32)
    wxyz = jnp.pad(W0[:, 0:3].T, ((0, 5), (0, 0))).astype(f32)
    params = jnp.zeros((8, 128), f32)
    params = params.at[0, 0:64].set(g0).at[1, 0:64].set(b0)
    params = params.at[2, 0:64].set(g1).at[3, 0:64].set(b1)
    params = params.at[4, 0:128].set(g2).at[5, 0:128].set(b2)

    pooled = _mlp(g3, cg, w0p, w1p, w2p, wxyz, params)

    new_xyz = jnp.transpose(cg[:, 0:3].reshape(B, S, 3), (0, 2, 1))
    new_points = jnp.transpose(pooled.reshape(B, S, MLP_CH[2]), (0, 2, 1))
    return (new_xyz, new_points, fps_idx)
